# baseline (speedup 1.0000x reference)
import functools

import jax
import jax.numpy as jnp
from jax import lax
from jax.experimental import pallas as pl
from jax.experimental.pallas import tpu as pltpu
from jax.experimental.pallas import tpu_sc as plsc

B = 16384
L = 50
D = 32
NC = 2
NS = 16
NW = NC * NS
COLS_W = B // NW
CC = 32
CHUNKS = COLS_W // CC
LG = 10
V = 16


def _gather_body(xt_hbm, table_hbm, out_hbm, idx0, idx1, rows0, rows1,
                 gsem0, gsem1, wsem0, wsem1):
    wid = lax.axis_index("s") * NC + lax.axis_index("c")
    col_base = wid * COLS_W
    idx = (idx0, idx1)
    rows = (rows0, rows1)
    gsem = (gsem0, gsem1)
    wsem = (wsem0, wsem1)
    iota = lax.broadcasted_iota(jnp.int32, (V,), 0)

    def load_idx(c, b):
        pltpu.sync_copy(
            xt_hbm.at[:, pl.ds(col_base + c * CC, CC)], idx[b]
        )

    def fire_gathers(b):
        def group(g, _):
            for j in range(LG):
                l = g * LG + j
                pltpu.async_copy(
                    table_hbm.at[idx[b].at[l]],
                    rows[b].at[l],
                    gsem[b],
                )
            return _
        lax.fori_loop(0, L // LG, group, None)

    def wait_gathers(b):
        pltpu.make_async_copy(
            out_hbm.at[:, :, pl.ds(0, CC)], rows[b], gsem[b]
        ).wait()

    def transpose_blocks(b):
        def one_l(l, _):
            il = jnp.full((V,), 0, jnp.int32) + l
            for i in (0, 1):
                for j in (0, 1):
                    if j < i:
                        continue
                    a_rows = [rows[b][l, V * i + k, pl.ds(V * j, V)]
                              for k in range(V)]
                    if i != j:
                        b_rows = [rows[b][l, V * j + k, pl.ds(V * i, V)]
                                  for k in range(V)]
                    for k in range(V):
                        plsc.store_scatter(
                            rows[b],
                            [il, V * j + iota,
                             jnp.full((V,), V * i + k, jnp.int32)],
                            a_rows[k],
                        )
                        if i != j:
                            plsc.store_scatter(
                                rows[b],
                                [il, V * i + iota,
                                 jnp.full((V,), V * j + k, jnp.int32)],
                                b_rows[k],
                            )
            return _
        lax.fori_loop(0, L, one_l, None)

    def write_async(c, b):
        pltpu.async_copy(
            rows[b], out_hbm.at[:, :, pl.ds(col_base + c * CC, CC)], wsem[b]
        )

    def wait_write(b):
        pltpu.make_async_copy(
            rows[b], out_hbm.at[:, :, pl.ds(0, CC)], wsem[b]
        ).wait()

    def pair(i, _):
        for h in (0, 1):
            c = 2 * i + h

            @pl.when(i > 0)
            def _wait_buf():
                wait_write(h)

            load_idx(c, h)
            fire_gathers(h)

            if h == 0:
                @pl.when(i > 0)
                def _drain_prev():
                    wait_gathers(1)
                    transpose_blocks(1)
                    write_async(c - 1, 1)
            else:
                wait_gathers(0)
                transpose_blocks(0)
                write_async(c - 1, 0)
        return _

    lax.fori_loop(0, CHUNKS // 2, pair, None)

    wait_gathers(1)
    transpose_blocks(1)
    pltpu.sync_copy(
        rows[1], out_hbm.at[:, :, pl.ds(col_base + (CHUNKS - 1) * CC, CC)]
    )
    wait_write(0)


@functools.partial(jax.jit, static_argnames=())
def kernel(x, table):
    xt = jnp.swapaxes(x, 0, 1).astype(jnp.int32)
    out = pl.kernel(
        _gather_body,
        out_type=jax.ShapeDtypeStruct((L, D, B), jnp.float32),
        mesh=plsc.VectorSubcoreMesh(core_axis_name="c", subcore_axis_name="s"),
        compiler_params=pltpu.CompilerParams(
            use_tc_tiling_on_sc=False, needs_layout_passes=False
        ),
        scratch_types=[
            pltpu.VMEM((L, CC), jnp.int32),
            pltpu.VMEM((L, CC), jnp.int32),
            pltpu.VMEM((L, CC, D), jnp.float32),
            pltpu.VMEM((L, CC, D), jnp.float32),
            pltpu.SemaphoreType.DMA,
            pltpu.SemaphoreType.DMA,
            pltpu.SemaphoreType.DMA,
            pltpu.SemaphoreType.DMA,
        ],
    )(xt, table.astype(jnp.float32))
    return jnp.transpose(out, (2, 0, 1))


# speedup vs baseline: 1.1003x; 1.1003x over previous
import functools

import jax
import jax.numpy as jnp
from jax import lax
from jax.experimental import pallas as pl
from jax.experimental.pallas import tpu as pltpu
from jax.experimental.pallas import tpu_sc as plsc

B = 16384
L = 50
D = 32
NC = 2
NS = 16
NW = NC * NS
COLS_W = B // NW
CC = 32
CHUNKS = COLS_W // CC
LG = 10
BJ = B // 128


def _detile_body(xt_hbm, xcube_hbm, buf):
    wid = lax.axis_index("s") * NC + lax.axis_index("c")
    pltpu.sync_copy(xt_hbm.at[:, pl.ds(wid * COLS_W, COLS_W)], buf)
    for q in range(COLS_W // 128):
        pltpu.sync_copy(
            buf.at[:, pl.ds(128 * q, 128)],
            xcube_hbm.at[:, wid * (COLS_W // 128) + q, :],
        )


def _gather_body(xcube_hbm, table_hbm, out_hbm, idx0, idx1, rows0, rows1,
                 gsem0, gsem1, wsem0, wsem1):
    wid = lax.axis_index("s") * NC + lax.axis_index("c")
    col_base = wid * COLS_W
    idx = (idx0, idx1)
    rows = (rows0, rows1)
    gsem = (gsem0, gsem1)
    wsem = (wsem0, wsem1)

    def load_idx(c, b):
        col = col_base + c * CC
        pltpu.sync_copy(
            xcube_hbm.at[:, col // 128, pl.ds(col % 128, CC)], idx[b]
        )

    def fire_gathers(b):
        def group(g, _):
            for j in range(LG):
                l = g * LG + j
                pltpu.async_copy(
                    table_hbm.at[idx[b].at[l]],
                    rows[b].at[l],
                    gsem[b],
                )
            return _
        lax.fori_loop(0, L // LG, group, None)

    def wait_gathers(b):
        pltpu.make_async_copy(
            out_hbm.at[:, pl.ds(0, CC)], rows[b], gsem[b]
        ).wait()

    def write_async(c, b):
        pltpu.async_copy(
            rows[b], out_hbm.at[:, pl.ds(col_base + c * CC, CC)], wsem[b]
        )

    def wait_write(b):
        pltpu.make_async_copy(
            rows[b], out_hbm.at[:, pl.ds(0, CC)], wsem[b]
        ).wait()

    def pair(i, _):
        for h in (0, 1):
            c = 2 * i + h

            @pl.when(i > 0)
            def _wait_buf():
                wait_write(h)

            load_idx(c, h)
            fire_gathers(h)

            if h == 0:
                @pl.when(i > 0)
                def _drain_prev():
                    wait_gathers(1)
                    write_async(c - 1, 1)
            else:
                wait_gathers(0)
                write_async(c - 1, 0)
        return _

    lax.fori_loop(0, CHUNKS // 2, pair, None)

    wait_gathers(1)
    pltpu.sync_copy(
        rows[1], out_hbm.at[:, pl.ds(col_base + (CHUNKS - 1) * CC, CC)]
    )
    wait_write(0)


_MESH = dict(core_axis_name="c", subcore_axis_name="s")


@functools.partial(jax.jit, static_argnames=())
def kernel(x, table):
    xt = jnp.swapaxes(x, 0, 1).astype(jnp.int32)
    xcube = pl.kernel(
        _detile_body,
        out_type=jax.ShapeDtypeStruct((L, BJ, 128), jnp.int32),
        mesh=plsc.VectorSubcoreMesh(**_MESH),
        scratch_types=[pltpu.VMEM((L, COLS_W), jnp.int32)],
    )(xt)
    out = pl.kernel(
        _gather_body,
        out_type=jax.ShapeDtypeStruct((L, B, D), jnp.float32),
        mesh=plsc.VectorSubcoreMesh(**_MESH),
        compiler_params=pltpu.CompilerParams(use_tc_tiling_on_sc=False),
        scratch_types=[
            pltpu.VMEM((L, CC), jnp.int32),
            pltpu.VMEM((L, CC), jnp.int32),
            pltpu.VMEM((L, CC, D), jnp.float32),
            pltpu.VMEM((L, CC, D), jnp.float32),
            pltpu.SemaphoreType.DMA,
            pltpu.SemaphoreType.DMA,
            pltpu.SemaphoreType.DMA,
            pltpu.SemaphoreType.DMA,
        ],
    )(xcube, table.astype(jnp.float32))
    return jnp.swapaxes(out, 0, 1)
